# baseline (speedup 1.0000x reference)
import functools

import jax
import jax.numpy as jnp
from jax import lax
from jax.experimental import pallas as pl
from jax.experimental.pallas import tpu as pltpu
from jax.experimental.pallas import tpu_sc as plsc

_NC = 2
_NS = 16
_NW = _NC * _NS
_LANES = 16


def _sc_embed(comp, lev, t, *, n_comp, n_lev, rows_per_tile, K):
    N = comp.shape[0]
    D = 1 + n_comp + n_lev
    nchunks = rows_per_tile // K
    mesh = plsc.VectorSubcoreMesh(core_axis_name="c", subcore_axis_name="s")

    @functools.partial(
        pl.kernel,
        out_type=jax.ShapeDtypeStruct((N * D,), jnp.float32),
        mesh=mesh,
        scratch_types=[
            pltpu.VMEM((K * D,), jnp.float32),
            pltpu.VMEM((K,), jnp.int32),
            pltpu.VMEM((K,), jnp.int32),
            pltpu.VMEM((K,), jnp.float32),
        ],
        compiler_params=pltpu.CompilerParams(needs_layout_passes=False),
    )
    def body(comp_hbm, lev_hbm, t_hbm, out_hbm, stage, comp_v, lev_v, t_v):
        wid = lax.axis_index("s") * _NC + lax.axis_index("c")
        tile_base = wid * rows_per_tile
        ones = jnp.full((_LANES,), 1.0, jnp.float32)
        zeros = jnp.zeros((_LANES,), jnp.float32)
        lane = lax.iota(jnp.int32, _LANES)

        def zero_body(i, carry):
            stage[pl.ds(i * _LANES, _LANES)] = zeros
            return carry

        lax.fori_loop(0, (K * D) // _LANES, zero_body, 0)

        def chunk_body(c, carry):
            base = tile_base + c * K
            pltpu.sync_copy(comp_hbm.at[pl.ds(base, K)], comp_v)
            pltpu.sync_copy(lev_hbm.at[pl.ds(base, K)], lev_v)
            pltpu.sync_copy(t_hbm.at[pl.ds(base, K)], t_v)

            def group(g, carry2):
                r0 = g * _LANES
                rowbase = (lane + r0) * D
                c16 = comp_v[pl.ds(r0, _LANES)]
                l16 = lev_v[pl.ds(r0, _LANES)]
                t16 = t_v[pl.ds(r0, _LANES)]
                plsc.store_scatter(stage, [rowbase], t16)
                plsc.store_scatter(stage, [rowbase + 1 + c16], ones,
                                   mask=c16 < n_comp)
                plsc.store_scatter(stage, [rowbase + (1 + n_comp) + l16], ones,
                                   mask=l16 < n_lev)
                return carry2

            lax.fori_loop(0, K // _LANES, group, 0)
            pltpu.sync_copy(stage, out_hbm.at[pl.ds(base * D, K * D)])

            def ungroup(g, carry2):
                r0 = g * _LANES
                rowbase = (lane + r0) * D
                c16 = comp_v[pl.ds(r0, _LANES)]
                l16 = lev_v[pl.ds(r0, _LANES)]
                plsc.store_scatter(stage, [rowbase + 1 + c16], zeros,
                                   mask=c16 < n_comp)
                plsc.store_scatter(stage, [rowbase + (1 + n_comp) + l16], zeros,
                                   mask=l16 < n_lev)
                return carry2

            lax.fori_loop(0, K // _LANES, ungroup, 0)
            return carry

        lax.fori_loop(0, nchunks, chunk_body, 0)

    return body(comp, lev, t)


def kernel(components, levels, time_elapsed, comp_table, level_table):
    n_comp = comp_table.shape[1]
    n_lev = level_table.shape[1]
    D = 1 + n_comp + n_lev
    B, L = components.shape
    N = B * L

    comp = components.reshape(N).astype(jnp.int32)
    lev = levels.reshape(N).astype(jnp.int32)
    t = time_elapsed.reshape(N)

    rows_per_tile = N // _NW
    K = 320
    assert rows_per_tile % K == 0

    out_flat = _sc_embed(comp, lev, t, n_comp=n_comp, n_lev=n_lev,
                         rows_per_tile=rows_per_tile, K=K)
    return out_flat.reshape(B, L, D)


# speedup vs baseline: 3.0545x; 3.0545x over previous
import functools

import jax
import jax.numpy as jnp
from jax import lax
from jax.experimental import pallas as pl


def _body(comp_ref, lev_ref, t_ref, out_ref, *, n_comp, n_lev):
    Rb, L, D = out_ref.shape
    i = lax.broadcasted_iota(jnp.int32, (Rb, L, D), 2)
    comp = comp_ref[...][:, :, None]
    lev = lev_ref[...][:, :, None]
    t = t_ref[...][:, :, None]
    one = jnp.float32(1.0)
    zero = jnp.float32(0.0)
    c_val = jnp.where(i == comp + 1, one, zero)
    l_val = jnp.where(i == lev + (n_comp + 1), one, zero)
    onehot = jnp.where(i <= n_comp, c_val, l_val)
    out_ref[...] = jnp.where(i == 0, t, onehot)


def kernel(components, levels, time_elapsed, comp_table, level_table):
    n_comp = comp_table.shape[1]
    n_lev = level_table.shape[1]
    D = 1 + n_comp + n_lev
    B, L = components.shape

    comp = components.astype(jnp.int32)
    lev = levels.astype(jnp.int32)

    Rb = 64
    assert B % Rb == 0
    body = functools.partial(_body, n_comp=n_comp, n_lev=n_lev)

    out = pl.pallas_call(
        body,
        grid=(B // Rb,),
        in_specs=[
            pl.BlockSpec((Rb, L), lambda g: (g, 0)),
            pl.BlockSpec((Rb, L), lambda g: (g, 0)),
            pl.BlockSpec((Rb, L), lambda g: (g, 0)),
        ],
        out_specs=pl.BlockSpec((Rb, L, D), lambda g: (g, 0, 0)),
        out_shape=jax.ShapeDtypeStruct((B, L, D), jnp.float32),
    )(comp, lev, time_elapsed)
    return out
